# Initial kernel scaffold; baseline (speedup 1.0000x reference)
import jax
import jax.numpy as jnp
from jax.experimental import pallas as pl


def kernel(x, edge_index, actions, Wm1, bm1, Wm2, bm2, Wm3, bm3, Wa1, ba1, Wa2, ba2, Wu1, bu1, Wu2, bu2, Wu3, bu3, Wact, bact, Wh1, bh1, Wh2, bh2, Wq, bq):
    raise NotImplementedError("write your pallas kernel here")


# speedup vs baseline: 9.5990x; 9.5990x over previous
import functools

import jax
import jax.numpy as jnp
from jax import lax
from jax.experimental import pallas as pl
from jax.experimental.pallas import tpu as pltpu
from jax.experimental.pallas import tpu_sc as plsc

N_NODES = 10000
N_AGENTS = 1000
N_EDGES = 320000
D = 128
H1 = 256
NC = 2
NS = 16
NW = NC * NS
EPW = N_EDGES // NW
BLK = 512
CAP = 10240
RCH = 64
NBW = CAP // BLK


def _relu(v):
    return jnp.maximum(v, 0.0)


def _stage1_body(x_ref, wm1_ref, bm1_ref, xws_ref, xwd_ref):
    x = x_ref[...]
    xws_ref[...] = jnp.dot(x, wm1_ref[:D, :], preferred_element_type=jnp.float32)
    xwd_ref[...] = (
        jnp.dot(x_ref[:N_AGENTS, :], wm1_ref[D:, :], preferred_element_type=jnp.float32)
        + bm1_ref[...]
    )


def _stage1(x, wm1, bm1r):
    return pl.pallas_call(
        _stage1_body,
        out_shape=[
            jax.ShapeDtypeStruct((N_NODES, H1), jnp.float32),
            jax.ShapeDtypeStruct((N_AGENTS, H1), jnp.float32),
        ],
    )(x, wm1, bm1r)


def _sc_body(src_hbm, dst_hbm, xws_hbm, xwd_hbm,
             pres_hbm, pred_hbm, dstc_hbm, cnt_hbm,
             src_v, dst_v, csrc_v, cdst_v, bufa, bufb, cvec, sema, semb):
    wid = lax.axis_index("s") * NC + lax.axis_index("c")
    ebase = wid * EPW
    pltpu.sync_copy(src_hbm.at[pl.ds(ebase, EPW)], src_v)
    pltpu.sync_copy(dst_hbm.at[pl.ds(ebase, EPW)], dst_v)

    one16 = jnp.ones((16,), jnp.int32)
    zer16 = jnp.zeros((16,), jnp.int32)

    def cbody(i, cnt):
        s = src_v[pl.ds(i * 16, 16)]
        dv = dst_v[pl.ds(i * 16, 16)]
        m = dv < N_AGENTS
        mi = jnp.where(m, one16, zer16)
        csum = plsc.cumsum(mi)
        pos = cnt + csum - 1
        plsc.store_scatter(csrc_v, [pos], s, mask=m)
        plsc.store_scatter(cdst_v, [pos], dv, mask=m)
        return cnt + csum[15]

    cnt = lax.fori_loop(0, EPW // 16, cbody, jnp.int32(0))

    z16 = jnp.zeros((16,), jnp.int32)
    for t in range(RCH // 16):
        csrc_v[pl.ds(cnt + t * 16, 16)] = z16
        cdst_v[pl.ds(cnt + t * 16, 16)] = z16

    cvec[...] = jnp.zeros((16,), jnp.int32) + cnt
    pltpu.sync_copy(cvec, cnt_hbm.at[wid])

    obase = wid * CAP
    ncg = (cnt + (RCH - 1)) // RCH

    def gbody(c, carry):
        off = c * RCH
        ca = pltpu.async_copy(xws_hbm.at[csrc_v.at[pl.ds(off, RCH)]], bufa, sema)
        cb = pltpu.async_copy(xwd_hbm.at[cdst_v.at[pl.ds(off, RCH)]], bufb, semb)
        ca.wait()
        cb.wait()
        pltpu.sync_copy(bufa, pres_hbm.at[pl.ds(obase + off, RCH)])
        pltpu.sync_copy(bufb, pred_hbm.at[pl.ds(obase + off, RCH)])
        pltpu.sync_copy(cdst_v.at[pl.ds(off, RCH)], dstc_hbm.at[pl.ds(obase + off, RCH)])
        return carry

    lax.fori_loop(0, ncg, gbody, 0)


def _stage2(src, dst, xws, xwd):
    mesh = plsc.VectorSubcoreMesh(core_axis_name="c", subcore_axis_name="s")
    f = functools.partial(
        pl.kernel,
        mesh=mesh,
        out_type=[
            jax.ShapeDtypeStruct((NW * CAP, H1), jnp.float32),
            jax.ShapeDtypeStruct((NW * CAP, H1), jnp.float32),
            jax.ShapeDtypeStruct((NW * CAP,), jnp.int32),
            jax.ShapeDtypeStruct((NW, 16), jnp.int32),
        ],
        scratch_types=[
            pltpu.VMEM((EPW,), jnp.int32),
            pltpu.VMEM((EPW,), jnp.int32),
            pltpu.VMEM((CAP,), jnp.int32),
            pltpu.VMEM((CAP,), jnp.int32),
            pltpu.VMEM((RCH, H1), jnp.float32),
            pltpu.VMEM((RCH, H1), jnp.float32),
            pltpu.VMEM((16,), jnp.int32),
            pltpu.SemaphoreType.DMA,
            pltpu.SemaphoreType.DMA,
        ],
        compiler_params=pltpu.CompilerParams(needs_layout_passes=False),
    )(_sc_body)
    return f(src, dst, xws, xwd)


def _critic_body(cnt_s, pres_hbm, pred_hbm, dstc_hbm,
                 x_ref, act_ref,
                 wm2_ref, bm2_ref, wm3_ref, bm3_ref,
                 wa1_ref, ba1_ref, wa2_ref, ba2_ref,
                 wu1_ref, bu1_ref, wu2_ref, bu2_ref, wu3_ref, bu3_ref,
                 wact_ref, bact_ref, wh1_ref, bh1_ref, wh2_ref, bh2_ref,
                 wq_ref, bq_ref,
                 out_ref,
                 sbuf, dbuf, dstv, aggr, sem1, sem2, sem3):
    aggr[...] = jnp.zeros_like(aggr)

    def tile_body(t, carry):
        cnt = cnt_s[t, 0]
        nb = (cnt + (BLK - 1)) // BLK

        def blk_body(b, carry2):
            row0 = t * CAP + b * BLK
            cs = pltpu.make_async_copy(pres_hbm.at[pl.ds(row0, BLK)], sbuf, sem1)
            cd = pltpu.make_async_copy(pred_hbm.at[pl.ds(row0, BLK)], dbuf, sem2)
            ci = pltpu.make_async_copy(dstc_hbm.at[t * NBW + b], dstv, sem3)
            cs.start()
            cd.start()
            ci.start()
            cs.wait()
            cd.wait()
            ci.wait()
            rows = b * BLK + lax.broadcasted_iota(jnp.int32, (BLK, 1), 0)
            m1 = jnp.where(rows < cnt, _relu(sbuf[...] + dbuf[...]), 0.0)
            m2 = _relu(jnp.dot(m1, wm2_ref[...], preferred_element_type=jnp.float32)
                       + bm2_ref[...])
            m3 = (jnp.dot(m2, wm3_ref[...], preferred_element_type=jnp.float32)
                  + bm3_ref[...])
            cols = b * BLK + lax.broadcasted_iota(jnp.int32, (1, BLK), 1)
            dsel = jnp.where(cols < cnt, dstv[...].reshape(1, BLK), N_AGENTS)
            oh = (lax.broadcasted_iota(jnp.int32, (N_AGENTS, BLK), 0) == dsel
                  ).astype(jnp.float32)
            aggr[...] = aggr[...] + jnp.dot(oh, m3, preferred_element_type=jnp.float32)
            return carry2

        return lax.fori_loop(0, nb, blk_body, carry)

    lax.fori_loop(0, NW, tile_body, 0)

    ag = aggr[...]
    a = _relu(jnp.dot(ag, wa1_ref[...], preferred_element_type=jnp.float32) + ba1_ref[...])
    a = _relu(jnp.dot(a, wa2_ref[...], preferred_element_type=jnp.float32) + ba2_ref[...])
    h = _relu(jnp.dot(x_ref[...], wu1_ref[:D, :], preferred_element_type=jnp.float32)
              + jnp.dot(a, wu1_ref[D:, :], preferred_element_type=jnp.float32)
              + bu1_ref[...])
    h = _relu(jnp.dot(h, wu2_ref[...], preferred_element_type=jnp.float32) + bu2_ref[...])
    h = jnp.dot(h, wu3_ref[...], preferred_element_type=jnp.float32) + bu3_ref[...]
    ap = _relu(jnp.dot(act_ref[...], wact_ref[...], preferred_element_type=jnp.float32)
               + bact_ref[...])
    z = _relu(jnp.dot(h, wh1_ref[:D, :], preferred_element_type=jnp.float32)
              + jnp.dot(ap, wh1_ref[D:, :], preferred_element_type=jnp.float32)
              + bh1_ref[...])
    z = _relu(jnp.dot(z, wh2_ref[...], preferred_element_type=jnp.float32) + bh2_ref[...])
    q = jnp.sum(z * wq_ref[...], axis=1, keepdims=True) + bq_ref[...]
    out_ref[...] = jnp.broadcast_to(q, (N_AGENTS, D))


def _stage3(counts, pres, pred, dstc2, x_a, actions, weights):
    in_specs = [pl.BlockSpec(memory_space=pltpu.SMEM),
                pl.BlockSpec(memory_space=pl.ANY),
                pl.BlockSpec(memory_space=pl.ANY),
                pl.BlockSpec(memory_space=pl.ANY)]
    in_specs += [pl.BlockSpec(memory_space=pltpu.VMEM)] * (2 + len(weights))
    return pl.pallas_call(
        _critic_body,
        out_shape=jax.ShapeDtypeStruct((N_AGENTS, D), jnp.float32),
        in_specs=in_specs,
        out_specs=pl.BlockSpec(memory_space=pltpu.VMEM),
        scratch_shapes=[
            pltpu.VMEM((BLK, H1), jnp.float32),
            pltpu.VMEM((BLK, H1), jnp.float32),
            pltpu.VMEM((BLK,), jnp.int32),
            pltpu.VMEM((N_AGENTS, D), jnp.float32),
            pltpu.SemaphoreType.DMA,
            pltpu.SemaphoreType.DMA,
            pltpu.SemaphoreType.DMA,
        ],
    )(counts, pres, pred, dstc2, x_a, actions, *weights)


def kernel(x, edge_index, actions,
           Wm1, bm1, Wm2, bm2, Wm3, bm3,
           Wa1, ba1, Wa2, ba2,
           Wu1, bu1, Wu2, bu2, Wu3, bu3,
           Wact, bact, Wh1, bh1, Wh2, bh2, Wq, bq):
    xws, xwd = _stage1(x, Wm1, bm1.reshape(1, -1))
    src = edge_index[0]
    dst = edge_index[1]
    pres, pred, dstc, counts = _stage2(src, dst, xws, xwd)
    dstc2 = dstc.reshape(NW * NBW, BLK)
    weights = (Wm2, bm2.reshape(1, -1), Wm3, bm3.reshape(1, -1),
               Wa1, ba1.reshape(1, -1), Wa2, ba2.reshape(1, -1),
               Wu1, bu1.reshape(1, -1), Wu2, bu2.reshape(1, -1),
               Wu3, bu3.reshape(1, -1),
               Wact, bact.reshape(1, -1),
               Wh1, bh1.reshape(1, -1), Wh2, bh2.reshape(1, -1),
               Wq.reshape(1, -1), bq.reshape(1, 1))
    out = _stage3(counts, pres, pred, dstc2, x[:N_AGENTS], actions, weights)
    return out[:, 0]


# speedup vs baseline: 13.2962x; 1.3852x over previous
import functools

import jax
import jax.numpy as jnp
from jax import lax
from jax.experimental import pallas as pl
from jax.experimental.pallas import tpu as pltpu
from jax.experimental.pallas import tpu_sc as plsc

N_NODES = 10000
N_AGENTS = 1000
N_EDGES = 320000
D = 128
H1 = 256
NC = 2
NS = 16
NW = NC * NS
EPW = N_EDGES // NW
BLK = 512
CAP = 10240
RCH = 64
NBW = CAP // BLK
MAXB = NW * NBW


def _relu(v):
    return jnp.maximum(v, 0.0)


def _stage1_body(x_ref, wm1_ref, bm1_ref, xws_ref, xwd_ref):
    x = x_ref[...]
    xws_ref[...] = jnp.dot(x, wm1_ref[:D, :], preferred_element_type=jnp.float32)
    xwd_ref[...] = (
        jnp.dot(x_ref[:N_AGENTS, :], wm1_ref[D:, :], preferred_element_type=jnp.float32)
        + bm1_ref[...]
    )


def _stage1(x, wm1, bm1r):
    return pl.pallas_call(
        _stage1_body,
        out_shape=[
            jax.ShapeDtypeStruct((N_NODES, H1), jnp.float32),
            jax.ShapeDtypeStruct((N_AGENTS, H1), jnp.float32),
        ],
    )(x, wm1, bm1r)


def _sc_body(ei_hbm, xws_hbm, xwd_hbm,
             pres_hbm, pred_hbm, dstc_hbm, cnt_hbm,
             src_v, dst_v, csrc_v, cdst_v, bufa, bufb, cvec, sema, semb):
    wid = lax.axis_index("s") * NC + lax.axis_index("c")
    ebase = wid * EPW
    pltpu.sync_copy(ei_hbm.at[pl.ds(ebase, EPW)], src_v)
    pltpu.sync_copy(ei_hbm.at[pl.ds(N_EDGES + ebase, EPW)], dst_v)

    one16 = jnp.ones((16,), jnp.int32)
    zer16 = jnp.zeros((16,), jnp.int32)

    def cbody(i, cnt):
        s = src_v[pl.ds(i * 16, 16)]
        dv = dst_v[pl.ds(i * 16, 16)]
        m = dv < N_AGENTS
        mi = jnp.where(m, one16, zer16)
        csum = plsc.cumsum(mi)
        pos = cnt + csum - 1
        plsc.store_scatter(csrc_v, [pos], s, mask=m)
        plsc.store_scatter(cdst_v, [pos], dv, mask=m)
        return cnt + csum[15]

    cnt = lax.fori_loop(0, EPW // 16, cbody, jnp.int32(0))

    for t in range(RCH // 16):
        csrc_v[pl.ds(cnt + t * 16, 16)] = zer16
        cdst_v[pl.ds(cnt + t * 16, 16)] = zer16

    cvec[...] = zer16 + cnt
    pltpu.sync_copy(cvec, cnt_hbm.at[wid])

    obase = wid * CAP
    ncg = (cnt + (RCH - 1)) // RCH

    def gbody(c, carry):
        off = c * RCH
        ca = pltpu.async_copy(xws_hbm.at[csrc_v.at[pl.ds(off, RCH)]], bufa, sema)
        cb = pltpu.async_copy(xwd_hbm.at[cdst_v.at[pl.ds(off, RCH)]], bufb, semb)
        ca.wait()
        cb.wait()
        pltpu.sync_copy(bufa, pres_hbm.at[pl.ds(obase + off, RCH)])
        pltpu.sync_copy(bufb, pred_hbm.at[pl.ds(obase + off, RCH)])
        pltpu.sync_copy(
            cdst_v.at[pl.ds(off, RCH)],
            dstc_hbm.at[(obase + off) // BLK, 0, pl.ds((obase + off) % BLK, RCH)],
        )
        return carry

    lax.fori_loop(0, ncg, gbody, 0)


def _stage2(edge_index, xws, xwd):
    mesh = plsc.VectorSubcoreMesh(core_axis_name="c", subcore_axis_name="s")
    f = functools.partial(
        pl.kernel,
        mesh=mesh,
        out_type=[
            jax.ShapeDtypeStruct((NW * CAP, H1), jnp.float32),
            jax.ShapeDtypeStruct((NW * CAP, H1), jnp.float32),
            jax.ShapeDtypeStruct((MAXB, 1, BLK), jnp.int32),
            jax.ShapeDtypeStruct((NW, 16), jnp.int32),
        ],
        scratch_types=[
            pltpu.VMEM((EPW,), jnp.int32),
            pltpu.VMEM((EPW,), jnp.int32),
            pltpu.VMEM((CAP,), jnp.int32),
            pltpu.VMEM((CAP,), jnp.int32),
            pltpu.VMEM((RCH, H1), jnp.float32),
            pltpu.VMEM((RCH, H1), jnp.float32),
            pltpu.VMEM((16,), jnp.int32),
            pltpu.SemaphoreType.DMA,
            pltpu.SemaphoreType.DMA,
        ],
        compiler_params=pltpu.CompilerParams(needs_layout_passes=False),
    )(_sc_body)
    return f(edge_index, xws, xwd)


def _critic_body(cnt_s, pres_hbm, pred_hbm, dstc_hbm,
                 x_ref, act_ref,
                 wm2_ref, bm2_ref, wm3_ref, bm3_ref,
                 wa1_ref, ba1_ref, wa2_ref, ba2_ref,
                 wu1_ref, bu1_ref, wu2_ref, bu2_ref, wu3_ref, bu3_ref,
                 wact_ref, bact_ref, wh1_ref, bh1_ref, wh2_ref, bh2_ref,
                 wq_ref, bq_ref,
                 out_ref,
                 sbuf, dbuf, dstv, aggr, rows_s, drow_s, vlim_s, sem):
    aggr[...] = jnp.zeros_like(aggr)

    def touter(t, idx):
        cnt = cnt_s[t, 0]
        nb = (cnt + (BLK - 1)) // BLK

        def binner(b, idx2):
            rows_s[idx2] = t * CAP + b * BLK
            drow_s[idx2] = t * NBW + b
            vlim_s[idx2] = cnt - b * BLK
            return idx2 + 1

        return lax.fori_loop(0, nb, binner, idx)

    total = lax.fori_loop(0, NW, touter, jnp.int32(0))

    def issue(k):
        s = lax.rem(k, 2)
        row0 = pl.multiple_of(rows_s[k], BLK)
        dr = drow_s[k]
        pltpu.make_async_copy(pres_hbm.at[pl.ds(row0, BLK)], sbuf.at[s], sem.at[s, 0]).start()
        pltpu.make_async_copy(pred_hbm.at[pl.ds(row0, BLK)], dbuf.at[s], sem.at[s, 1]).start()
        pltpu.make_async_copy(dstc_hbm.at[dr], dstv.at[s], sem.at[s, 2]).start()

    def wait(k):
        s = lax.rem(k, 2)
        pltpu.make_async_copy(pres_hbm.at[pl.ds(0, BLK)], sbuf.at[s], sem.at[s, 0]).wait()
        pltpu.make_async_copy(pred_hbm.at[pl.ds(0, BLK)], dbuf.at[s], sem.at[s, 1]).wait()
        pltpu.make_async_copy(dstc_hbm.at[0], dstv.at[s], sem.at[s, 2]).wait()

    @pl.when(total > 0)
    def _():
        issue(jnp.int32(0))

    def kbody(k, carry):
        @pl.when(k + 1 < total)
        def _():
            issue(k + 1)

        wait(k)
        s = lax.rem(k, 2)
        vlim = vlim_s[k]
        rows = lax.broadcasted_iota(jnp.int32, (BLK, 1), 0)
        m1 = jnp.where(rows < vlim, _relu(sbuf[s] + dbuf[s]), 0.0)
        m2 = _relu(jnp.dot(m1, wm2_ref[...], preferred_element_type=jnp.float32)
                   + bm2_ref[...])
        m3 = (jnp.dot(m2, wm3_ref[...], preferred_element_type=jnp.float32)
              + bm3_ref[...])
        cols = lax.broadcasted_iota(jnp.int32, (1, BLK), 1)
        dsel = jnp.where(cols < vlim, dstv[s], N_AGENTS)
        oh = (lax.broadcasted_iota(jnp.int32, (N_AGENTS, BLK), 0) == dsel
              ).astype(jnp.float32)
        aggr[...] = aggr[...] + jnp.dot(oh, m3, preferred_element_type=jnp.float32)
        return carry

    lax.fori_loop(0, total, kbody, 0)

    ag = aggr[...]
    a = _relu(jnp.dot(ag, wa1_ref[...], preferred_element_type=jnp.float32) + ba1_ref[...])
    a = _relu(jnp.dot(a, wa2_ref[...], preferred_element_type=jnp.float32) + ba2_ref[...])
    h = _relu(jnp.dot(x_ref[...], wu1_ref[:D, :], preferred_element_type=jnp.float32)
              + jnp.dot(a, wu1_ref[D:, :], preferred_element_type=jnp.float32)
              + bu1_ref[...])
    h = _relu(jnp.dot(h, wu2_ref[...], preferred_element_type=jnp.float32) + bu2_ref[...])
    h = jnp.dot(h, wu3_ref[...], preferred_element_type=jnp.float32) + bu3_ref[...]
    ap = _relu(jnp.dot(act_ref[...], wact_ref[...], preferred_element_type=jnp.float32)
               + bact_ref[...])
    z = _relu(jnp.dot(h, wh1_ref[:D, :], preferred_element_type=jnp.float32)
              + jnp.dot(ap, wh1_ref[D:, :], preferred_element_type=jnp.float32)
              + bh1_ref[...])
    z = _relu(jnp.dot(z, wh2_ref[...], preferred_element_type=jnp.float32) + bh2_ref[...])
    q = jnp.sum(z * wq_ref[...], axis=1, keepdims=True) + bq_ref[...]
    out_ref[...] = q


def _stage3(counts, pres, pred, dstc2, x, actions, weights):
    in_specs = [pl.BlockSpec(memory_space=pltpu.SMEM),
                pl.BlockSpec(memory_space=pl.ANY),
                pl.BlockSpec(memory_space=pl.ANY),
                pl.BlockSpec(memory_space=pl.ANY),
                pl.BlockSpec(memory_space=pltpu.VMEM)]
    in_specs += [pl.BlockSpec(memory_space=pltpu.VMEM)] * (1 + len(weights))
    return pl.pallas_call(
        _critic_body,
        out_shape=jax.ShapeDtypeStruct((N_AGENTS, 1), jnp.float32),
        in_specs=in_specs,
        out_specs=pl.BlockSpec(memory_space=pltpu.VMEM),
        scratch_shapes=[
            pltpu.VMEM((2, BLK, H1), jnp.float32),
            pltpu.VMEM((2, BLK, H1), jnp.float32),
            pltpu.VMEM((2, 1, BLK), jnp.int32),
            pltpu.VMEM((N_AGENTS, D), jnp.float32),
            pltpu.SMEM((MAXB,), jnp.int32),
            pltpu.SMEM((MAXB,), jnp.int32),
            pltpu.SMEM((MAXB,), jnp.int32),
            pltpu.SemaphoreType.DMA((2, 3)),
        ],
    )(counts, pres, pred, dstc2, x, actions, *weights)


def kernel(x, edge_index, actions,
           Wm1, bm1, Wm2, bm2, Wm3, bm3,
           Wa1, ba1, Wa2, ba2,
           Wu1, bu1, Wu2, bu2, Wu3, bu3,
           Wact, bact, Wh1, bh1, Wh2, bh2, Wq, bq):
    xws, xwd = _stage1(x, Wm1, bm1.reshape(1, -1))
    pres, pred, dstc2, counts = _stage2(edge_index.reshape(-1), xws, xwd)
    weights = (Wm2, bm2.reshape(1, -1), Wm3, bm3.reshape(1, -1),
               Wa1, ba1.reshape(1, -1), Wa2, ba2.reshape(1, -1),
               Wu1, bu1.reshape(1, -1), Wu2, bu2.reshape(1, -1),
               Wu3, bu3.reshape(1, -1),
               Wact, bact.reshape(1, -1),
               Wh1, bh1.reshape(1, -1), Wh2, bh2.reshape(1, -1),
               Wq.reshape(1, -1), bq.reshape(1, 1))
    out = _stage3(counts, pres, pred, dstc2, x[:N_AGENTS], actions, weights)
    return out.reshape(N_AGENTS)
